# baseline (speedup 1.0000x reference)
import functools

import jax
import jax.numpy as jnp
import numpy as np
from jax import lax
from jax.experimental import pallas as pl
from jax.experimental.pallas import tpu as pltpu
from jax.experimental.pallas import tpu_sc as plsc

B = 5
N0 = 10000
POS = 16
HID = 64
HEADS = 2
KC = 10
TAU = 1.0
EPS = 1e-15

LANES = 16
CHUNK = 128
NTILES = 16


def _npad(n):
    return ((n + 127) // 128) * 128


SLAB = 8
EGRAN = NTILES * CHUNK * SLAB * 2


@functools.cache
def _spmm_fn(npad, e_pad, cw=32):
    srows = e_pad // CHUNK
    spt = srows // NTILES
    t2 = spt // (SLAB * 2)
    wrpt = npad // NTILES
    zrpt = (npad + CHUNK) // NTILES
    mesh = plsc.VectorSubcoreMesh(core_axis_name="c", subcore_axis_name="s")

    @functools.partial(
        pl.kernel,
        out_type=jax.ShapeDtypeStruct((2 * npad, cw), jnp.float32),
        mesh=mesh,
        scratch_types=[
            pltpu.VMEM((SLAB, CHUNK), jnp.int32),
            pltpu.VMEM((SLAB, CHUNK), jnp.int32),
            pltpu.VMEM((SLAB, CHUNK), jnp.int32),
            pltpu.VMEM((SLAB, CHUNK), jnp.int32),
            pltpu.VMEM((SLAB, CHUNK, cw), jnp.float32),
            pltpu.VMEM((SLAB, CHUNK, cw), jnp.float32),
            pltpu.VMEM_SHARED((npad + CHUNK, cw), jnp.float32),
            pltpu.SemaphoreType.DMA,
            pltpu.SemaphoreType.DMA,
            pltpu.SemaphoreType.DMA,
            pltpu.SemaphoreType.DMA,
        ],
        compiler_params=pltpu.CompilerParams(use_tc_tiling_on_sc=False),
    )
    def k(xs_lo, xs_hi, src2d, route2d, zeros_hbm, out_hbm,
          sidx0, sidx1, ridx0, ridx1, rows0, rows1, acc, gs0, gs1, ss0, ss1):
        c = lax.axis_index("c")
        s = lax.axis_index("s")
        pltpu.sync_copy(zeros_hbm.at[pl.ds(0, zrpt)], acc.at[pl.ds(s * zrpt, zrpt)])
        plsc.subcore_barrier()
        r0 = s * spt
        bufs = ((sidx0, ridx0, rows0, gs0, ss0), (sidx1, ridx1, rows1, gs1, ss1))

        def run(xs):
            def body(tp, carry):
                for b in (0, 1):
                    sidx, ridx, rows, gs, ss = bufs[b]

                    @pl.when(tp > 0)
                    def _():
                        for j in range(SLAB):
                            pltpu.make_async_copy(
                                rows.at[j], acc.at[ridx.at[j]], ss).wait()

                    base = r0 + (tp * 2 + b) * SLAB
                    pltpu.sync_copy(src2d.at[pl.ds(base, SLAB)], sidx)
                    pltpu.sync_copy(route2d.at[pl.ds(base, SLAB)], ridx)
                    descs = [pltpu.async_copy(xs.at[sidx.at[j]], rows.at[j], gs)
                             for j in range(SLAB)]
                    for d in descs:
                        d.wait()
                    for j in range(SLAB):
                        pltpu.async_copy(rows.at[j], acc.at[ridx.at[j]], ss,
                                         add=True)
                return carry

            lax.fori_loop(0, t2, body, 0)
            for b in (0, 1):
                sidx, ridx, rows, gs, ss = bufs[b]
                for j in range(SLAB):
                    pltpu.make_async_copy(rows.at[j], acc.at[ridx.at[j]], ss).wait()

        @pl.when(c == 0)
        def _():
            run(xs_lo)

        @pl.when(c == 1)
        def _():
            run(xs_hi)

        plsc.subcore_barrier()
        pltpu.sync_copy(acc.at[pl.ds(s * wrpt, wrpt)],
                        out_hbm.at[pl.ds(c * npad + s * wrpt, wrpt)])

    return k


def _spmm(xs, src, route, n):
    npad = _npad(n)
    e_pad = src.shape[0]
    pad_rows = npad - n
    cw = 16 if (npad + CHUNK) * 32 > 900_000 else 32
    zrpt = (npad + CHUNK) // NTILES
    zeros = jnp.zeros((zrpt, cw), jnp.float32)
    src2d = src.reshape(-1, CHUNK)
    route2d = route.reshape(-1, CHUNK)
    fn = _spmm_fn(npad, e_pad, cw)
    parts = []
    for q in range(0, 64, 2 * cw):
        lo = jnp.pad(xs[:, q:q + cw], ((0, pad_rows), (0, 0)))
        hi = jnp.pad(xs[:, q + cw:q + 2 * cw], ((0, pad_rows), (0, 0)))
        out = fn(lo, hi, src2d, route2d, zeros)
        parts.append(out[:n])
        parts.append(out[npad:npad + n])
    return jnp.concatenate(parts, axis=1)


@functools.cache
def _spmm_split_fn(npad, nsplit, e_pad):
    srows = e_pad // CHUNK
    spt = srows // NTILES
    t2 = spt // (SLAB * 2)
    wrpt = nsplit // NTILES
    zrpt = (nsplit + CHUNK) // NTILES
    mesh = plsc.VectorSubcoreMesh(core_axis_name="c", subcore_axis_name="s")

    @functools.partial(
        pl.kernel,
        out_type=jax.ShapeDtypeStruct((2 * nsplit, 32), jnp.float32),
        mesh=mesh,
        scratch_types=[
            pltpu.VMEM((SLAB, CHUNK), jnp.int32),
            pltpu.VMEM((SLAB, CHUNK), jnp.int32),
            pltpu.VMEM((SLAB, CHUNK), jnp.int32),
            pltpu.VMEM((SLAB, CHUNK), jnp.int32),
            pltpu.VMEM((SLAB, CHUNK, 32), jnp.float32),
            pltpu.VMEM((SLAB, CHUNK, 32), jnp.float32),
            pltpu.VMEM_SHARED((nsplit + CHUNK, 32), jnp.float32),
            pltpu.SemaphoreType.DMA,
            pltpu.SemaphoreType.DMA,
            pltpu.SemaphoreType.DMA,
            pltpu.SemaphoreType.DMA,
        ],
        compiler_params=pltpu.CompilerParams(use_tc_tiling_on_sc=False),
    )
    def k(xs_h, slo2d, rlo2d, shi2d, rhi2d, zeros_hbm, out_hbm,
          sidx0, sidx1, ridx0, ridx1, rows0, rows1, acc, gs0, gs1, ss0, ss1):
        c = lax.axis_index("c")
        s = lax.axis_index("s")
        pltpu.sync_copy(zeros_hbm.at[pl.ds(0, zrpt)], acc.at[pl.ds(s * zrpt, zrpt)])
        plsc.subcore_barrier()
        r0 = s * spt
        bufs = ((sidx0, ridx0, rows0, gs0, ss0), (sidx1, ridx1, rows1, gs1, ss1))

        def run(s2d, r2d):
            def body(tp, carry):
                for b in (0, 1):
                    sidx, ridx, rows, gs, ss = bufs[b]

                    @pl.when(tp > 0)
                    def _():
                        for j in range(SLAB):
                            pltpu.make_async_copy(
                                rows.at[j], acc.at[ridx.at[j]], ss).wait()

                    base = r0 + (tp * 2 + b) * SLAB
                    pltpu.sync_copy(s2d.at[pl.ds(base, SLAB)], sidx)
                    pltpu.sync_copy(r2d.at[pl.ds(base, SLAB)], ridx)
                    descs = [pltpu.async_copy(xs_h.at[sidx.at[j]], rows.at[j], gs)
                             for j in range(SLAB)]
                    for d in descs:
                        d.wait()
                    for j in range(SLAB):
                        pltpu.async_copy(rows.at[j], acc.at[ridx.at[j]], ss,
                                         add=True)
                return carry

            lax.fori_loop(0, t2, body, 0)
            for b in (0, 1):
                sidx, ridx, rows, gs, ss = bufs[b]
                for j in range(SLAB):
                    pltpu.make_async_copy(rows.at[j], acc.at[ridx.at[j]], ss).wait()

        @pl.when(c == 0)
        def _():
            run(slo2d, rlo2d)

        @pl.when(c == 1)
        def _():
            run(shi2d, rhi2d)

        plsc.subcore_barrier()
        pltpu.sync_copy(acc.at[pl.ds(s * wrpt, wrpt)],
                        out_hbm.at[pl.ds(c * nsplit + s * wrpt, wrpt)])

    return k


def _spmm_split(xs, slo, rlo, shi, rhi, n, nsplit):
    npad = _npad(n)
    e_pad = slo.shape[0]
    pad_rows = npad - n
    zeros = jnp.zeros(((nsplit + CHUNK) // NTILES, 32), jnp.float32)
    fn = _spmm_split_fn(npad, nsplit, e_pad)
    s2 = (slo.reshape(-1, CHUNK), rlo.reshape(-1, CHUNK),
          shi.reshape(-1, CHUNK), rhi.reshape(-1, CHUNK))
    parts = []
    for h in range(2):
        xs_h = jnp.pad(xs[:, 32 * h:32 * h + 32], ((0, pad_rows), (0, 0)))
        out = fn(xs_h, *s2, zeros)
        parts.append(out[:n])
    return jnp.concatenate(parts, axis=1)


@functools.cache
def _prep_fn(np_prev, nd_new, e_pad, nsplit=0):
    ept = e_pad // (2 * NTILES)
    nslab = ept // 1024
    dzpt = (nd_new + CHUNK) // NTILES
    dwpt = nd_new // NTILES
    trash = nd_new
    mesh = plsc.VectorSubcoreMesh(core_axis_name="c", subcore_axis_name="s")

    @functools.partial(
        pl.kernel,
        out_type=(
            jax.ShapeDtypeStruct((e_pad,), jnp.int32),
            jax.ShapeDtypeStruct((e_pad,), jnp.int32),
            jax.ShapeDtypeStruct((e_pad,), jnp.float32),
            jax.ShapeDtypeStruct((e_pad,), jnp.int32),
            jax.ShapeDtypeStruct((2 * nd_new,), jnp.float32),
        ) + ((jax.ShapeDtypeStruct((e_pad,), jnp.int32),
              jax.ShapeDtypeStruct((e_pad,), jnp.int32),
              jax.ShapeDtypeStruct((e_pad,), jnp.int32),
              jax.ShapeDtypeStruct((e_pad,), jnp.int32),
              ) if nsplit else ()),
        mesh=mesh,
        scratch_types=[
            pltpu.VMEM((np_prev,), jnp.int32),
            pltpu.VMEM((1024,), jnp.int32),
            pltpu.VMEM((1024,), jnp.int32),
            pltpu.VMEM((1024,), jnp.float32),
            pltpu.VMEM((1024,), jnp.int32),
            pltpu.VMEM((1024,), jnp.int32),
            pltpu.VMEM((1024,), jnp.float32),
            pltpu.VMEM((1024,), jnp.int32),
            pltpu.VMEM((CHUNK,), jnp.int32),
            pltpu.VMEM((1024,), jnp.int32),
            pltpu.VMEM((1024,), jnp.int32),
            pltpu.VMEM((1024,), jnp.int32),
            pltpu.VMEM((1024,), jnp.int32),
            pltpu.VMEM_SHARED((nd_new + CHUNK,), jnp.float32),
        ],
        compiler_params=pltpu.CompilerParams(use_tc_tiling_on_sc=False,
                                             needs_layout_passes=False),
    )
    def k(map_hbm, src_hbm, dst_hbm, w_hbm, zeros_hbm,
          so_hbm, do_hbm, wo_hbm, ro_hbm, deg_hbm, *rest):
        if nsplit:
            (slo_hbm, rlo_hbm, shi_hbm, rhi_hbm,
             mapv, sbuf, dbuf, wbuf, snb, dnb, wnb, rtb, rti,
             slb, rlb, shb, rhb, dacc) = rest
        else:
            (mapv, sbuf, dbuf, wbuf, snb, dnb, wnb, rtb, rti,
             slb, rlb, shb, rhb, dacc) = rest
        c = lax.axis_index("c")
        s = lax.axis_index("s")
        pltpu.sync_copy(zeros_hbm.at[pl.ds(0, dzpt)], dacc.at[pl.ds(s * dzpt, dzpt)])
        pltpu.sync_copy(map_hbm, mapv)
        plsc.subcore_barrier()
        ebase = (s * 2 + c) * ept

        def body(t, carry):
            base = ebase + t * 1024
            pltpu.sync_copy(src_hbm.at[pl.ds(base, 1024)], sbuf)
            pltpu.sync_copy(dst_hbm.at[pl.ds(base, 1024)], dbuf)
            pltpu.sync_copy(w_hbm.at[pl.ds(base, 1024)], wbuf)
            for j in range(8):
                for g in range(8):
                    sl = pl.ds(j * 128 + g * 16, 16)
                    s16 = sbuf[sl]
                    d16 = dbuf[sl]
                    w16 = wbuf[sl]
                    ns = plsc.load_gather(mapv, [s16])
                    nd = plsc.load_gather(mapv, [d16])
                    valid = (ns >= 0) & (nd >= 0)
                    wn = jnp.where(valid, w16, 0.0)
                    rt = jnp.where(valid & (w16 > 0.0), nd,
                                   jnp.full((LANES,), trash, jnp.int32))
                    snb[sl] = jnp.where(valid, ns, 0)
                    dnb[sl] = jnp.where(valid, nd, 0)
                    wnb[sl] = wn
                    rtb[sl] = rt
                    rti[pl.ds(g * 16, 16)] = rt
                    if nsplit:
                        mem = valid & (w16 > 0.0)
                        lo = mem & (nd < nsplit)
                        hi = mem & (nd >= nsplit)
                        tl = jnp.full((LANES,), nsplit, jnp.int32)
                        slb[sl] = jnp.where(lo, ns, 0)
                        rlb[sl] = jnp.where(lo, nd, tl)
                        shb[sl] = jnp.where(hi, ns, 0)
                        rhb[sl] = jnp.where(hi, nd - nsplit, tl)
                pltpu.sync_copy(wnb.at[pl.ds(j * 128, CHUNK)],
                                dacc.at[rti], add=True)
            pltpu.sync_copy(snb, so_hbm.at[pl.ds(base, 1024)])
            pltpu.sync_copy(dnb, do_hbm.at[pl.ds(base, 1024)])
            pltpu.sync_copy(wnb, wo_hbm.at[pl.ds(base, 1024)])
            pltpu.sync_copy(rtb, ro_hbm.at[pl.ds(base, 1024)])
            if nsplit:
                pltpu.sync_copy(slb, slo_hbm.at[pl.ds(base, 1024)])
                pltpu.sync_copy(rlb, rlo_hbm.at[pl.ds(base, 1024)])
                pltpu.sync_copy(shb, shi_hbm.at[pl.ds(base, 1024)])
                pltpu.sync_copy(rhb, rhi_hbm.at[pl.ds(base, 1024)])
            return carry

        lax.fori_loop(0, nslab, body, 0)
        plsc.subcore_barrier()
        pltpu.sync_copy(dacc.at[pl.ds(s * dwpt, dwpt)],
                        deg_hbm.at[pl.ds(c * nd_new + s * dwpt, dwpt)])

    return k


def _prep(src, dst, w, mapping, n_new, nsplit=0):
    np_prev = _npad(mapping.shape[0])
    nd_new = _npad(n_new)
    e_pad = src.shape[0]
    map_pad = jnp.pad(mapping.astype(jnp.int32), (0, np_prev - mapping.shape[0]),
                      constant_values=-1)
    zeros = jnp.zeros(((nd_new + CHUNK) // NTILES,), jnp.float32)
    outs = _prep_fn(np_prev, nd_new, e_pad, nsplit)(map_pad, src, dst, w, zeros)
    so, do, wo, ro, degf = outs[:5]
    deg = degf[:n_new] + degf[nd_new:nd_new + n_new] + 1.0
    return (so, do, wo, ro, deg) + tuple(outs[5:])


def _head_body(g_ref, w1_ref, b1_ref, w2_ref, b2_ref, out_ref):
    g = g_ref[...]
    h = jnp.maximum(jnp.dot(g, w1_ref[...]) + b1_ref[...], 0.0)
    out_ref[...] = jnp.dot(h, w2_ref[...]) + b2_ref[...]


def _head(g, params):
    return pl.pallas_call(
        _head_body,
        out_shape=jax.ShapeDtypeStruct((B, 2), jnp.float32),
    )(g, params['clf_W1'], params['clf_b1'][None, :],
      params['clf_W2'], params['clf_b2'][None, :])


def _gcn(x, estate, W, b):
    src, route, deg = estate[0], estate[3], estate[4]
    n = x.shape[0]
    xw = x @ W
    dinv = deg ** -0.5
    xs = dinv[:, None] * xw
    if len(estate) > 5:
        slo, rlo, shi, rhi = estate[5:]
        msg = _spmm_split(xs, slo, rlo, shi, rhi, n, _npad(n // 2))
    else:
        msg = _spmm(xs, src, route, n)
    out = dinv[:, None] * msg + (1.0 / deg)[:, None] * xw
    return out + b


def _block(pb, x, estate, pos, n, first, last):
    src, dst, w = estate[0], estate[1], estate[2]
    z = x
    h = (x @ pb['proj_W']).reshape(-1, POS, HID)
    h = (h * pos[:, :, None]).sum(axis=1)
    h = _gcn(h, estate, pb['gcn_W'], pb['gcn_b'])
    mu = h.mean(axis=0)
    var = h.var(axis=0)
    h = (h - mu) / jnp.sqrt(var + 1e-5) * pb['bn_g'] + pb['bn_b']
    h = jax.nn.relu(h)
    if first:
        z = z @ pb['res_W'] + pb['res_b']
    h = h + z
    p = pb['pool_p']
    score = jnp.tanh((h @ p) / jnp.linalg.norm(p))
    k = int(np.ceil(0.5 * n))
    _, top_i = jax.lax.top_k(score.reshape(B, n), k)
    perm = (top_i + (jnp.arange(B, dtype=top_i.dtype) * n)[:, None]).reshape(-1)
    h_new = h[perm] * score[perm][:, None]
    old_n = B * n
    if last:
        new_estate = None
    else:
        mapping = jnp.full((old_n,), -1, jnp.int32).at[perm].set(
            jnp.arange(B * k, dtype=jnp.int32))
        new_estate = _prep(src, dst, w, mapping, B * k)
    return h_new, new_estate, pos[perm], k


def kernel(x, edge_index, batch, params):
    E = edge_index.shape[1]
    e_pad = ((E + EGRAN - 1) // EGRAN) * EGRAN
    src = jnp.zeros((e_pad,), jnp.int32).at[:E].set(edge_index[0].astype(jnp.int32))
    dst = jnp.zeros((e_pad,), jnp.int32).at[:E].set(edge_index[1].astype(jnp.int32))
    w = jnp.zeros((e_pad,), jnp.float32).at[:E].set(1.0)
    x_input = x.reshape(B, -1)
    pos_idx = jnp.tile(jnp.arange(N0), B)
    pos = jax.nn.softmax(params['pos_emb'][pos_idx], axis=-1)
    h = x
    n = N0
    estate = _prep(src, dst, w, jnp.arange(B * N0, dtype=jnp.int32), B * N0,
                   nsplit=_npad(B * N0 // 2))
    for i, pb in enumerate(params['blocks']):
        h, estate, pos, n = _block(pb, h, estate, pos, n, i == 0, i == 2)
    d2 = ((params['mem_k'].reshape(HEADS * KC, HID)[:, None, :] - h[None, :, :]) ** 2).sum(-1)
    d2 = (1.0 + d2 / TAU) ** (-(TAU + 1.0) / 2.0)
    d2 = d2.reshape(HEADS, KC, B, n).transpose(2, 3, 0, 1)
    S = d2 / d2.sum(axis=-1, keepdims=True)
    S = jnp.einsum('h,bnhk->bnk', params['mem_conv'], S)
    S = jax.nn.softmax(S, axis=-1)
    xd = h.reshape(B, n, HID)
    xp = jnp.einsum('bnk,bnd->bkd', S, xd) @ params['mem_lin']
    P = S ** 2 / S.sum(axis=1, keepdims=True)
    denom = P.sum(axis=2, keepdims=True)
    denom = jnp.where(S.sum(axis=2, keepdims=True) == 0.0, 1.0, denom)
    P = P / denom
    Pc = jnp.clip(P, EPS)
    kl = (Pc * (jnp.log(Pc) - jnp.log(jnp.clip(S, EPS)))).sum() / B
    g = xp.reshape(B, -1) @ params['fc1_W'] + params['fc1_b']
    g = g + x_input @ params['gres_W'] + params['gres_b']
    logits = _head(g, params)
    return logits, kl


# speedup vs baseline: 1.9775x; 1.9775x over previous
import functools

import jax
import jax.numpy as jnp
import numpy as np
from jax import lax
from jax.experimental import pallas as pl
from jax.experimental.pallas import tpu as pltpu
from jax.experimental.pallas import tpu_sc as plsc

B = 5
N0 = 10000
POS = 16
HID = 64
HEADS = 2
KC = 10
TAU = 1.0
EPS = 1e-15

LANES = 16
CHUNK = 128
NTILES = 16


def _npad(n):
    return ((n + 127) // 128) * 128


SLAB = 8
EGRAN = NTILES * CHUNK * SLAB * 2


@functools.cache
def _spmm_fn(npad, e_pad, cw=32):
    srows = e_pad // CHUNK
    spt = srows // NTILES
    t2 = spt // (SLAB * 2)
    wrpt = npad // NTILES
    zrpt = (npad + CHUNK) // NTILES
    mesh = plsc.VectorSubcoreMesh(core_axis_name="c", subcore_axis_name="s")

    @functools.partial(
        pl.kernel,
        out_type=jax.ShapeDtypeStruct((2 * npad, cw), jnp.float32),
        mesh=mesh,
        scratch_types=[
            pltpu.VMEM((SLAB, CHUNK), jnp.int32),
            pltpu.VMEM((SLAB, CHUNK), jnp.int32),
            pltpu.VMEM((SLAB, CHUNK), jnp.int32),
            pltpu.VMEM((SLAB, CHUNK), jnp.int32),
            pltpu.VMEM((SLAB, CHUNK, cw), jnp.float32),
            pltpu.VMEM((SLAB, CHUNK, cw), jnp.float32),
            pltpu.VMEM_SHARED((npad + CHUNK, cw), jnp.float32),
            pltpu.SemaphoreType.DMA,
            pltpu.SemaphoreType.DMA,
            pltpu.SemaphoreType.DMA,
            pltpu.SemaphoreType.DMA,
        ],
        compiler_params=pltpu.CompilerParams(use_tc_tiling_on_sc=False),
    )
    def k(xs_lo, xs_hi, src2d, route2d, zeros_hbm, out_hbm,
          sidx0, sidx1, ridx0, ridx1, rows0, rows1, acc, gs0, gs1, ss0, ss1):
        c = lax.axis_index("c")
        s = lax.axis_index("s")
        pltpu.sync_copy(zeros_hbm.at[pl.ds(0, zrpt)], acc.at[pl.ds(s * zrpt, zrpt)])
        plsc.subcore_barrier()
        r0 = s * spt
        bufs = ((sidx0, ridx0, rows0, gs0, ss0), (sidx1, ridx1, rows1, gs1, ss1))

        def run(xs):
            def body(tp, carry):
                for b in (0, 1):
                    sidx, ridx, rows, gs, ss = bufs[b]

                    @pl.when(tp > 0)
                    def _():
                        for j in range(SLAB):
                            pltpu.make_async_copy(
                                rows.at[j], acc.at[ridx.at[j]], ss).wait()

                    base = r0 + (tp * 2 + b) * SLAB
                    pltpu.sync_copy(src2d.at[pl.ds(base, SLAB)], sidx)
                    pltpu.sync_copy(route2d.at[pl.ds(base, SLAB)], ridx)
                    descs = [pltpu.async_copy(xs.at[sidx.at[j]], rows.at[j], gs)
                             for j in range(SLAB)]
                    for d in descs:
                        d.wait()
                    for j in range(SLAB):
                        pltpu.async_copy(rows.at[j], acc.at[ridx.at[j]], ss,
                                         add=True)
                return carry

            lax.fori_loop(0, t2, body, 0)
            for b in (0, 1):
                sidx, ridx, rows, gs, ss = bufs[b]
                for j in range(SLAB):
                    pltpu.make_async_copy(rows.at[j], acc.at[ridx.at[j]], ss).wait()

        @pl.when(c == 0)
        def _():
            run(xs_lo)

        @pl.when(c == 1)
        def _():
            run(xs_hi)

        plsc.subcore_barrier()
        pltpu.sync_copy(acc.at[pl.ds(s * wrpt, wrpt)],
                        out_hbm.at[pl.ds(c * npad + s * wrpt, wrpt)])

    return k


def _spmm(xs, src, route, n):
    npad = _npad(n)
    e_pad = src.shape[0]
    pad_rows = npad - n
    cw = 16 if (npad + CHUNK) * 32 > 900_000 else 32
    zrpt = (npad + CHUNK) // NTILES
    zeros = jnp.zeros((zrpt, cw), jnp.float32)
    src2d = src.reshape(-1, CHUNK)
    route2d = route.reshape(-1, CHUNK)
    fn = _spmm_fn(npad, e_pad, cw)
    parts = []
    for q in range(0, 64, 2 * cw):
        lo = jnp.pad(xs[:, q:q + cw], ((0, pad_rows), (0, 0)))
        hi = jnp.pad(xs[:, q + cw:q + 2 * cw], ((0, pad_rows), (0, 0)))
        out = fn(lo, hi, src2d, route2d, zeros)
        parts.append(out[:n])
        parts.append(out[npad:npad + n])
    return jnp.concatenate(parts, axis=1)


@functools.cache
def _prep_fn(np_prev, nd_new, e_pad):
    ept = e_pad // (2 * NTILES)
    nslab = ept // 1024
    dzpt = (nd_new + CHUNK) // NTILES
    dwpt = nd_new // NTILES
    trash = nd_new
    mesh = plsc.VectorSubcoreMesh(core_axis_name="c", subcore_axis_name="s")

    @functools.partial(
        pl.kernel,
        out_type=(
            jax.ShapeDtypeStruct((e_pad,), jnp.int32),
            jax.ShapeDtypeStruct((e_pad,), jnp.int32),
            jax.ShapeDtypeStruct((e_pad,), jnp.float32),
            jax.ShapeDtypeStruct((e_pad,), jnp.int32),
            jax.ShapeDtypeStruct((2 * nd_new,), jnp.float32),
        ),
        mesh=mesh,
        scratch_types=[
            pltpu.VMEM((np_prev,), jnp.int32),
            pltpu.VMEM((1024,), jnp.int32),
            pltpu.VMEM((1024,), jnp.int32),
            pltpu.VMEM((1024,), jnp.float32),
            pltpu.VMEM((1024,), jnp.int32),
            pltpu.VMEM((1024,), jnp.int32),
            pltpu.VMEM((1024,), jnp.float32),
            pltpu.VMEM((1024,), jnp.int32),
            pltpu.VMEM((CHUNK,), jnp.int32),
            pltpu.VMEM_SHARED((nd_new + CHUNK,), jnp.float32),
        ],
        compiler_params=pltpu.CompilerParams(use_tc_tiling_on_sc=False,
                                             needs_layout_passes=False),
    )
    def k(map_hbm, src_hbm, dst_hbm, w_hbm, zeros_hbm,
          so_hbm, do_hbm, wo_hbm, ro_hbm, deg_hbm,
          mapv, sbuf, dbuf, wbuf, snb, dnb, wnb, rtb, rti, dacc):
        c = lax.axis_index("c")
        s = lax.axis_index("s")
        pltpu.sync_copy(zeros_hbm.at[pl.ds(0, dzpt)], dacc.at[pl.ds(s * dzpt, dzpt)])
        pltpu.sync_copy(map_hbm, mapv)
        plsc.subcore_barrier()
        ebase = (s * 2 + c) * ept

        def body(t, carry):
            base = ebase + t * 1024
            pltpu.sync_copy(src_hbm.at[pl.ds(base, 1024)], sbuf)
            pltpu.sync_copy(dst_hbm.at[pl.ds(base, 1024)], dbuf)
            pltpu.sync_copy(w_hbm.at[pl.ds(base, 1024)], wbuf)
            for j in range(8):
                for g in range(8):
                    sl = pl.ds(j * 128 + g * 16, 16)
                    s16 = sbuf[sl]
                    d16 = dbuf[sl]
                    w16 = wbuf[sl]
                    ns = plsc.load_gather(mapv, [s16])
                    nd = plsc.load_gather(mapv, [d16])
                    valid = (ns >= 0) & (nd >= 0)
                    wn = jnp.where(valid, w16, 0.0)
                    rt = jnp.where(valid & (w16 > 0.0), nd,
                                   jnp.full((LANES,), trash, jnp.int32))
                    snb[sl] = jnp.where(valid, ns, 0)
                    dnb[sl] = jnp.where(valid, nd, 0)
                    wnb[sl] = wn
                    rtb[sl] = rt
                    rti[pl.ds(g * 16, 16)] = rt
                pltpu.sync_copy(wnb.at[pl.ds(j * 128, CHUNK)],
                                dacc.at[rti], add=True)
            pltpu.sync_copy(snb, so_hbm.at[pl.ds(base, 1024)])
            pltpu.sync_copy(dnb, do_hbm.at[pl.ds(base, 1024)])
            pltpu.sync_copy(wnb, wo_hbm.at[pl.ds(base, 1024)])
            pltpu.sync_copy(rtb, ro_hbm.at[pl.ds(base, 1024)])
            return carry

        lax.fori_loop(0, nslab, body, 0)
        plsc.subcore_barrier()
        pltpu.sync_copy(dacc.at[pl.ds(s * dwpt, dwpt)],
                        deg_hbm.at[pl.ds(c * nd_new + s * dwpt, dwpt)])

    return k


def _prep(src, dst, w, mapping, n_new):
    np_prev = _npad(mapping.shape[0])
    nd_new = _npad(n_new)
    e_pad = src.shape[0]
    map_pad = jnp.pad(mapping.astype(jnp.int32), (0, np_prev - mapping.shape[0]),
                      constant_values=-1)
    zeros = jnp.zeros(((nd_new + CHUNK) // NTILES,), jnp.float32)
    so, do, wo, ro, degf = _prep_fn(np_prev, nd_new, e_pad)(
        map_pad, src, dst, w, zeros)
    deg = degf[:n_new] + degf[nd_new:nd_new + n_new] + 1.0
    return so, do, wo, ro, deg


def _head_body(g_ref, w1_ref, b1_ref, w2_ref, b2_ref, out_ref):
    g = g_ref[...]
    h = jnp.maximum(jnp.dot(g, w1_ref[...]) + b1_ref[...], 0.0)
    out_ref[...] = jnp.dot(h, w2_ref[...]) + b2_ref[...]


def _head(g, params):
    return pl.pallas_call(
        _head_body,
        out_shape=jax.ShapeDtypeStruct((B, 2), jnp.float32),
    )(g, params['clf_W1'], params['clf_b1'][None, :],
      params['clf_W2'], params['clf_b2'][None, :])


def _gcn(x, src, route, deg, W, b):
    n = x.shape[0]
    xw = x @ W
    dinv = deg ** -0.5
    xs = dinv[:, None] * xw
    msg = _spmm(xs, src, route, n)
    out = dinv[:, None] * msg + (1.0 / deg)[:, None] * xw
    return out + b


def _block(pb, x, estate, pos, n, first, last):
    src, dst, w, route, deg = estate
    z = x
    h = (x @ pb['proj_W']).reshape(-1, POS, HID)
    h = (h * pos[:, :, None]).sum(axis=1)
    h = _gcn(h, src, route, deg, pb['gcn_W'], pb['gcn_b'])
    mu = h.mean(axis=0)
    var = h.var(axis=0)
    h = (h - mu) / jnp.sqrt(var + 1e-5) * pb['bn_g'] + pb['bn_b']
    h = jax.nn.relu(h)
    if first:
        z = z @ pb['res_W'] + pb['res_b']
    h = h + z
    p = pb['pool_p']
    score = jnp.tanh((h @ p) / jnp.linalg.norm(p))
    k = int(np.ceil(0.5 * n))
    _, top_i = jax.lax.top_k(score.reshape(B, n), k)
    perm = (top_i + (jnp.arange(B, dtype=top_i.dtype) * n)[:, None]).reshape(-1)
    h_new = h[perm] * score[perm][:, None]
    old_n = B * n
    if last:
        new_estate = None
    else:
        mapping = jnp.full((old_n,), -1, jnp.int32).at[perm].set(
            jnp.arange(B * k, dtype=jnp.int32))
        new_estate = _prep(src, dst, w, mapping, B * k)
    return h_new, new_estate, pos[perm], k


def kernel(x, edge_index, batch, params):
    E = edge_index.shape[1]
    e_pad = ((E + EGRAN - 1) // EGRAN) * EGRAN
    src = jnp.zeros((e_pad,), jnp.int32).at[:E].set(edge_index[0].astype(jnp.int32))
    dst = jnp.zeros((e_pad,), jnp.int32).at[:E].set(edge_index[1].astype(jnp.int32))
    w = jnp.zeros((e_pad,), jnp.float32).at[:E].set(1.0)
    x_input = x.reshape(B, -1)
    pos_idx = jnp.tile(jnp.arange(N0), B)
    pos = jax.nn.softmax(params['pos_emb'][pos_idx], axis=-1)
    h = x
    n = N0
    estate = _prep(src, dst, w, jnp.arange(B * N0, dtype=jnp.int32), B * N0)
    for i, pb in enumerate(params['blocks']):
        h, estate, pos, n = _block(pb, h, estate, pos, n, i == 0, i == 2)
    d2 = ((params['mem_k'].reshape(HEADS * KC, HID)[:, None, :] - h[None, :, :]) ** 2).sum(-1)
    d2 = (1.0 + d2 / TAU) ** (-(TAU + 1.0) / 2.0)
    d2 = d2.reshape(HEADS, KC, B, n).transpose(2, 3, 0, 1)
    S = d2 / d2.sum(axis=-1, keepdims=True)
    S = jnp.einsum('h,bnhk->bnk', params['mem_conv'], S)
    S = jax.nn.softmax(S, axis=-1)
    xd = h.reshape(B, n, HID)
    xp = jnp.einsum('bnk,bnd->bkd', S, xd) @ params['mem_lin']
    P = S ** 2 / S.sum(axis=1, keepdims=True)
    denom = P.sum(axis=2, keepdims=True)
    denom = jnp.where(S.sum(axis=2, keepdims=True) == 0.0, 1.0, denom)
    P = P / denom
    Pc = jnp.clip(P, EPS)
    kl = (Pc * (jnp.log(Pc) - jnp.log(jnp.clip(S, EPS)))).sum() / B
    g = xp.reshape(B, -1) @ params['fc1_W'] + params['fc1_b']
    g = g + x_input @ params['gres_W'] + params['gres_b']
    logits = _head(g, params)
    return logits, kl
